# baseline (speedup 1.0000x reference)
import functools

import jax
import jax.numpy as jnp
from jax import lax
from jax.experimental import pallas as pl
from jax.experimental.pallas import tpu as pltpu
from jax.experimental.pallas import tpu_sc as plsc

f32 = jnp.float32
i32 = jnp.int32

_N = 50000
_NBUS = 500
_NGRAPH = 100
_E = 800000
_LAYERS = 10

_NS = 16
_PTILE = 3200
_NPAD = _PTILE * _NS
_NROW = _NPAD // 128
_AGROWS = 408
_MARGIN = 512
_W = _PTILE + _MARGIN
_WROWS = 40
_THB = 50048

_CROWS = 8
_NCHUNK = 49
_RPT = _CROWS * _NCHUNK
_ROWS = _RPT * _NS
_EPAD = _ROWS * 128
_ERRW = (_LAYERS + 1) * 16

_TBO = 0
_OUO = 3712
_EAO = 6912


def _sc_body(x0h, x1h, dph, s1h, d1h, w1h, s2h, d2h, w2h,
             outh, errh,
             th_sh, ag_sh, erra_sh,
             big1, accum, agw, pw, invdw, refx, zb,
             rowix, rowix16,
             srcb0, dstb0, wb0, errb, errt,
             sem_l, sem_s):
    wid = lax.axis_index("s")
    lo = pl.multiple_of(wid * _PTILE, 8)
    sw = pl.multiple_of(jnp.maximum(lo - _MARGIN, 0), 8)
    own = lo - sw
    r_lo = wid * 25
    rtop = jnp.maximum(r_lo - 4, 0)
    rw0 = pl.multiple_of((rtop // 8) * 8, 8)
    drow = rtop - rw0
    r0own = pl.multiple_of((r_lo // 8) * 8, 8)
    dro = r_lo - r0own
    tbase = wid * _RPT

    pltpu.sync_copy(x0h.at[pl.ds(sw, _W)], pw)
    pltpu.sync_copy(x1h.at[pl.ds(sw, _W)], big1.at[pl.ds(0, _W)])

    def pinit(v, _):
        sl = pl.ds(v * 16, 16)
        pw[sl] = pw[sl] - big1[sl]
        return 0
    lax.fori_loop(0, _W // 16, pinit, 0)

    pltpu.sync_copy(dph.at[pl.ds(sw, _W)], big1.at[pl.ds(0, _W)])

    def dinit(v, _):
        sl = pl.ds(v * 16, 16)
        d = big1[sl] * 100.0
        nz = d != 0.0
        dsafe = jnp.where(nz, d, 1.0)
        invdw[sl] = jnp.where(nz, 1.0 / dsafe, 0.0)
        return 0
    lax.fori_loop(0, _W // 16, dinit, 0)

    def minit(v, _):
        sl = pl.ds(v * 16, 16)
        i = lo + v * 16 + lax.iota(i32, 16)
        g = ((i.astype(f32) + 0.5) * (1.0 / 500.0)).astype(i32)
        g = jnp.minimum(g, _NGRAPH - 1)
        refx[sl] = g * _NBUS - sw
        return 0
    lax.fori_loop(0, _PTILE // 16, minit, 0)

    for r in range(8):
        for o in range(8):
            zb[r, pl.ds(o * 16, 16)] = jnp.zeros((16,), f32)
    for r in range(3):
        for o in range(8):
            rowix[r, pl.ds(o * 16, 16)] = (r * 128 + o * 16
                                           + lax.iota(i32, 16))
    rowix16[...] = _NROW - 16 + lax.iota(i32, 16)

    def zacc(r, _):
        for o in range(8):
            accum[r, pl.ds(o * 16, 16)] = jnp.zeros((16,), f32)
        return 0
    lax.fori_loop(0, _NROW, zacc, 0)

    def einit(kk, _):
        errb[pl.ds(kk * 16, 16)] = jnp.zeros((16,), f32)
        return 0
    lax.fori_loop(0, _LAYERS + 1, einit, 0)

    def zero_ag_blocks():
        zbase = pl.multiple_of(wid * 32, 8)

        @pl.when(wid < 12)
        def _():
            for b in range(4):
                pltpu.sync_copy(zb, ag_sh.at[pl.ds(zbase + b * 8, 8)])

        @pl.when(wid == 12)
        def _():
            for b in range(3):
                pltpu.sync_copy(zb, ag_sh.at[pl.ds(zbase + b * 8, 8)])
    zero_ag_blocks()
    plsc.subcore_barrier()

    def edge_pass(sh, dh, wh):
        pltpu.sync_copy(th_sh.at[pl.ds(0, _THB)], big1)

        def load(c, sb, db, wbx):
            rb = pl.multiple_of(tbase + c * _CROWS, 8)
            pltpu.async_copy(sh.at[pl.ds(rb, _CROWS)], sb, sem_l)
            pltpu.async_copy(dh.at[pl.ds(rb, _CROWS)], db, sem_l)
            pltpu.async_copy(wh.at[pl.ds(rb, _CROWS)], wbx, sem_l)

        def drain_loads():
            pltpu.make_async_copy(s1h.at[pl.ds(0, _CROWS)], srcb0, sem_l
                                  ).wait()
            pltpu.make_async_copy(s1h.at[pl.ds(0, _CROWS)], dstb0, sem_l
                                  ).wait()
            pltpu.make_async_copy(w1h.at[pl.ds(0, _CROWS)], wb0, sem_l
                                  ).wait()

        def compute(sb, db, wbx):
            def crow(r, _):
                for o in range(8):
                    sl = pl.ds(o * 16, 16)
                    sv = sb[r, sl]
                    dv = db[r, sl]
                    wv = wbx[r, sl]
                    th = plsc.load_gather(big1, [sv])
                    row = jnp.right_shift(dv, 7)
                    col = jnp.bitwise_and(dv, 127)
                    plsc.addupdate_scatter(accum, [row, col], th * wv)
                return 0
            lax.fori_loop(0, _CROWS, crow, 0)

        def chunk(c, _):
            load(c, srcb0, dstb0, wb0)
            drain_loads()
            compute(srcb0, dstb0, wb0)
            return 0
        lax.fori_loop(0, _NCHUNK, chunk, 0)

        rds = []
        for j in range(3):
            rds.append(pltpu.async_copy(accum.at[pl.ds(j * 128, 128)],
                                        ag_sh.at[rowix.at[j]], sem_s,
                                        add=True))
        rds.append(pltpu.async_copy(accum.at[pl.ds(_NROW - 16, 16)],
                                    ag_sh.at[rowix16], sem_s, add=True))
        for dsc in rds:
            dsc.wait()
        lax.fori_loop(0, _NROW, zacc, 0)

    def step(k, _):
        with jax.named_scope("ph_scatter1"):
            @pl.when(k > 0)
            def _():
                edge_pass(s1h, d1h, w1h)
            plsc.subcore_barrier()

        with jax.named_scope("ph_wincopy"):
            pltpu.sync_copy(ag_sh.at[pl.ds(rw0, _WROWS)], agw)
            plsc.subcore_barrier()
            zero_ag_blocks()

        ns_pw = jax.named_scope("ph_pointwise"); ns_pw.__enter__()

        def tcomp(r, _):
            for o in range(8):
                sl = pl.ds(r * 128 + o * 16, 16)
                big1[sl] = (pw[sl] - agw[drow + r, pl.ds(o * 16, 16)]) \
                    * invdw[sl]
            return 0
        lax.fori_loop(0, _W // 128, tcomp, 0)

        def ocomp(v, _):
            sl = pl.ds(v * 16, 16)
            t = big1[pl.ds(own + v * 16, 16)]
            tr = plsc.load_gather(big1, [refx[sl]])
            iv = invdw[pl.ds(own + v * 16, 16)]
            big1[pl.ds(_OUO + v * 16, 16)] = jnp.where(
                iv != 0.0, (t - tr) * 100.0, 0.0)
            return 0
        lax.fori_loop(0, _PTILE // 16, ocomp, 0)

        pltpu.sync_copy(big1.at[pl.ds(_OUO, _PTILE)],
                        th_sh.at[pl.ds(lo, _PTILE)])

        @pl.when(k == _LAYERS)
        def _():
            def fcomp(v, _):
                big1[pl.ds(v * 16, 16)] = big1[pl.ds(_OUO + v * 16, 16)] \
                    * 0.01
                return 0
            lax.fori_loop(0, _PTILE // 16, fcomp, 0)
            pltpu.sync_copy(big1.at[pl.ds(0, _PTILE)],
                            outh.at[pl.ds(lo, _PTILE)])
        plsc.subcore_barrier()
        ns_pw.__exit__(None, None, None)

        with jax.named_scope("ph_scatter2"):
            edge_pass(s2h, d2h, w2h)
            plsc.subcore_barrier()

        ns_er = jax.named_scope("ph_err"); ns_er.__enter__()
        pltpu.sync_copy(ag_sh.at[pl.ds(r0own, 32)], agw.at[pl.ds(0, 32)])

        def ecomp(r, acc):
            for o in range(8):
                e = pw[pl.ds(own + r * 128 + o * 16, 16)] \
                    - agw[dro + r, pl.ds(o * 16, 16)]
                acc = acc + jnp.abs(e)
            return acc
        acc = lax.fori_loop(0, 25, ecomp, jnp.zeros((16,), f32))
        errb[pl.ds(k * 16, 16)] = acc
        zero_ag_blocks()
        plsc.subcore_barrier()
        ns_er.__exit__(None, None, None)
        return 0
    lax.fori_loop(0, _LAYERS + 1, step, 0)

    pltpu.sync_copy(errb,
                    erra_sh.at[pl.ds(pl.multiple_of(wid * _ERRW, 8), _ERRW)])
    plsc.subcore_barrier()

    @pl.when(wid == 0)
    def _():
        pltpu.sync_copy(erra_sh, big1.at[pl.ds(_EAO, _NS * _ERRW)])

        def esum(kk, _):
            s = jnp.zeros((16,), f32)
            for t in range(_NS):
                s = s + big1[pl.ds(_EAO + t * _ERRW + kk * 16, 16)]
            errt[pl.ds(kk * 16, 16)] = s
            return 0
        lax.fori_loop(0, _LAYERS + 1, esum, 0)
        pltpu.sync_copy(errt, errh)


@functools.cache
def _build_sc_kernel():
  mesh = plsc.VectorSubcoreMesh(core_axis_name="c", subcore_axis_name="s",
                                num_cores=1, num_subcores=_NS)
  return functools.partial(
    pl.kernel,
    out_type=(jax.ShapeDtypeStruct((_NPAD,), f32),
              jax.ShapeDtypeStruct((_ERRW,), f32)),
    mesh=mesh,
    compiler_params=pltpu.CompilerParams(needs_layout_passes=False),
    scratch_types=[
        pltpu.VMEM_SHARED((_NPAD,), f32),
        pltpu.VMEM_SHARED((_AGROWS, 128), f32),
        pltpu.VMEM_SHARED((_NS * _ERRW,), f32),
        pltpu.VMEM((_THB,), f32),
        pltpu.VMEM((_NROW, 128), f32),
        pltpu.VMEM((_WROWS, 128), f32),
        pltpu.VMEM((_W,), f32),
        pltpu.VMEM((_W,), f32),
        pltpu.VMEM((_PTILE,), i32),
        pltpu.VMEM((8, 128), f32),
        pltpu.VMEM((3, 128), i32),
        pltpu.VMEM((16,), i32),
        pltpu.VMEM((_CROWS, 128), i32),
        pltpu.VMEM((_CROWS, 128), i32),
        pltpu.VMEM((_CROWS, 128), f32),
        pltpu.VMEM((_ERRW,), f32),
        pltpu.VMEM((_ERRW,), f32),
        pltpu.SemaphoreType.DMA,
        pltpu.SemaphoreType.DMA,
    ],
  )(_sc_body)


def kernel(x, y, edge_index_no_diag, edge_attr_no_diag, edge_index, edge_attr,
           ybus):
    del y
    x0 = jnp.pad(x[:, 0], (0, _NPAD - _N))
    x1 = jnp.pad(x[:, 1], (0, _NPAD - _N))
    eye = jnp.eye(_NBUS, dtype=f32)
    dg = (ybus * eye).sum(axis=2).reshape(-1)
    dp = jnp.pad(dg, (0, _NPAD - _N))

    def prep(ei, ea):
        s = jnp.pad(ei[0].astype(i32), (0, _EPAD - _E)).reshape(_ROWS, 128)
        d = jnp.pad(ei[1].astype(i32), (0, _EPAD - _E)).reshape(_ROWS, 128)
        w = jnp.pad(ea.astype(f32), (0, _EPAD - _E)).reshape(_ROWS, 128)
        return s, d, w

    s1, d1, w1 = prep(edge_index_no_diag, edge_attr_no_diag)
    s2, d2, w2 = prep(edge_index, edge_attr)

    outp, errs = _build_sc_kernel()(x0, x1, dp, s1, d1, w1, s2, d2, w2)
    out = outp[:_N].reshape(_N, 1)
    return (out, *(errs[k * 16:(k + 1) * 16].sum()
                   for k in range(_LAYERS + 1)))


# speedup vs baseline: 1.4965x; 1.4965x over previous
import functools

import jax
import jax.numpy as jnp
from jax import lax
from jax.experimental import pallas as pl
from jax.experimental.pallas import tpu as pltpu
from jax.experimental.pallas import tpu_sc as plsc

f32 = jnp.float32
i32 = jnp.int32

_N = 50000
_NBUS = 500
_NGRAPH = 100
_E = 800000
_LAYERS = 10

_NS = 16
_PTILE = 3136
_NPAD = _PTILE * _NS
_MARGIN = 512
_W = _PTILE + _MARGIN
_W2 = 3712

_CROWS = 56
_NCHUNK = 7
_RPT = _CROWS * _NCHUNK
_ROWS = _RPT * _NS
_EPAD = _ROWS * 128
_YTOT = _NGRAPH * _NBUS * _NBUS
_ERRW = (_LAYERS + 1) * 16


def _sc_body(x0h, x1h, dph, s1h, d1h, w1h, s2h, d2h, w2h,
             outh, errh,
             th_sh, ag_sh, erra_sh,
             pw, invdw, tb, agw, refx, outs, maskf, zb, threp,
             srcb, dstb0, dstb1, wb, msgb0, msgb1, errb, erracc, errt,
             sem_l, sem_s):
    wid = lax.axis_index("s")
    lo = pl.multiple_of(wid * _PTILE, 8)
    sw = pl.multiple_of(jnp.maximum(lo - _MARGIN, 0), 8)
    own = lo - sw
    tbase = wid * _RPT

    pltpu.sync_copy(x0h.at[pl.ds(sw, _W)], agw)
    pltpu.sync_copy(x1h.at[pl.ds(sw, _W)], tb)

    def pinit(v, _):
        pw[pl.ds(v * 16, 16)] = agw[pl.ds(v * 16, 16)] - tb[pl.ds(v * 16, 16)]
        return 0
    lax.fori_loop(0, _W // 16, pinit, 0)

    pltpu.sync_copy(dph.at[pl.ds(sw, _W)], tb)

    def dinit(v, _):
        d = tb[pl.ds(v * 16, 16)] * 100.0
        nz = d != 0.0
        dsafe = jnp.where(nz, d, 1.0)
        invdw[pl.ds(v * 16, 16)] = jnp.where(nz, 1.0 / dsafe, 0.0)
        return 0
    lax.fori_loop(0, _W // 16, dinit, 0)

    def minit(v, _):
        iv = invdw[pl.ds(own + v * 16, 16)]
        maskf[pl.ds(v * 16, 16)] = jnp.where(iv != 0.0, 1.0, 0.0)
        zb[pl.ds(v * 16, 16)] = jnp.zeros((16,), f32)
        i = lo + v * 16 + lax.iota(i32, 16)
        g = ((i.astype(f32) + 0.5) * (1.0 / 500.0)).astype(i32)
        g = jnp.minimum(g, _NGRAPH - 1)
        refx[pl.ds(v * 16, 16)] = g * _NBUS - sw
        return 0
    lax.fori_loop(0, _PTILE // 16, minit, 0)

    pltpu.sync_copy(zb, ag_sh.at[pl.ds(lo, _PTILE)])

    def einit(kk, _):
        errb[pl.ds(kk * 16, 16)] = jnp.zeros((16,), f32)
        return 0
    lax.fori_loop(0, _LAYERS + 1, einit, 0)
    plsc.subcore_barrier()

    _SCAT_BYTES = _CROWS * 128 * 4

    def edge_pass(sh, dh, wh):
        pltpu.sync_copy(th_sh, threp)

        def chunk(c, _):
            @pl.when(c > 0)
            def _():
                pltpu.make_async_copy(wh.at[pl.ds(0, _CROWS)], msgb0, sem_s
                                      ).wait()

            def do_chunk(dstx, msgx):
                rb = pl.multiple_of(tbase + c * _CROWS, 8)
                l1 = pltpu.async_copy(sh.at[pl.ds(rb, _CROWS)], srcb, sem_l)
                l2 = pltpu.async_copy(dh.at[pl.ds(rb, _CROWS)], dstx, sem_l)
                l3 = pltpu.async_copy(wh.at[pl.ds(rb, _CROWS)], wb, sem_l)
                l1.wait()
                l2.wait()
                l3.wait()

                def mrow(r, _):
                    for o in range(8):
                        sl = pl.ds(o * 16, 16)
                        th = plsc.load_gather(threp, [srcb[r, sl]])
                        msgx[r, sl] = th * wb[r, sl]
                    return 0
                lax.fori_loop(0, _CROWS, mrow, 0)
                for j in range(_CROWS):
                    pltpu.async_copy(msgx.at[j], ag_sh.at[dstx.at[j]], sem_s,
                                     add=True)

            @pl.when(c % 2 == 0)
            def _():
                do_chunk(dstb0, msgb0)

            @pl.when(c % 2 == 1)
            def _():
                do_chunk(dstb1, msgb1)
            return 0
        lax.fori_loop(0, _NCHUNK, chunk, 0)
        pltpu.make_async_copy(wh.at[pl.ds(0, _CROWS)], msgb0, sem_s).wait()

    def step(k, _):
        with jax.named_scope("ph_scatter1"):
            @pl.when(k > 0)
            def _():
                edge_pass(s1h, d1h, w1h)
            plsc.subcore_barrier()

        with jax.named_scope("ph_wincopy"):
            pltpu.sync_copy(ag_sh.at[pl.ds(sw, _W)], agw)
            plsc.subcore_barrier()
            pltpu.sync_copy(zb, ag_sh.at[pl.ds(lo, _PTILE)])

        ns_pw = jax.named_scope("ph_pointwise"); ns_pw.__enter__()
        def tcomp(v, _):
            sl = pl.ds(v * 16, 16)
            tb[sl] = (pw[sl] - agw[sl]) * invdw[sl]
            return 0
        lax.fori_loop(0, _W // 16, tcomp, 0)

        def ocomp(v, _):
            sl = pl.ds(v * 16, 16)
            t = tb[pl.ds(own + v * 16, 16)]
            tr = plsc.load_gather(tb, [refx[sl]])
            outs[sl] = (t - tr) * maskf[sl] * 100.0
            return 0
        lax.fori_loop(0, _PTILE // 16, ocomp, 0)

        pltpu.sync_copy(outs, th_sh.at[pl.ds(lo, _PTILE)])

        @pl.when(k == _LAYERS)
        def _():
            def fcomp(v, _):
                sl = pl.ds(v * 16, 16)
                tb[sl] = outs[sl] * 0.01
                return 0
            lax.fori_loop(0, _PTILE // 16, fcomp, 0)
            pltpu.sync_copy(tb.at[pl.ds(0, _PTILE)], outh.at[pl.ds(lo, _PTILE)])
        plsc.subcore_barrier()
        ns_pw.__exit__(None, None, None)

        with jax.named_scope("ph_scatter2"):
            edge_pass(s2h, d2h, w2h)
            plsc.subcore_barrier()

        ns_er = jax.named_scope("ph_err"); ns_er.__enter__()
        pltpu.sync_copy(ag_sh.at[pl.ds(lo, _PTILE)], agw.at[pl.ds(0, _PTILE)])

        def ecomp(v, acc):
            e = pw[pl.ds(own + v * 16, 16)] - agw[pl.ds(v * 16, 16)]
            return acc + jnp.abs(e)
        acc = lax.fori_loop(0, _PTILE // 16, ecomp, jnp.zeros((16,), f32))
        errb[pl.ds(k * 16, 16)] = acc
        pltpu.sync_copy(zb, ag_sh.at[pl.ds(lo, _PTILE)])
        plsc.subcore_barrier()
        ns_er.__exit__(None, None, None)
        return 0
    lax.fori_loop(0, _LAYERS + 1, step, 0)

    pltpu.sync_copy(errb, erra_sh.at[pl.ds(pl.multiple_of(wid * _ERRW, 8), _ERRW)])
    plsc.subcore_barrier()

    @pl.when(wid == 0)
    def _():
        pltpu.sync_copy(erra_sh, erracc)

        def esum(kk, _):
            s = jnp.zeros((16,), f32)
            for t in range(_NS):
                s = s + erracc[pl.ds(t * _ERRW + kk * 16, 16)]
            errt[pl.ds(kk * 16, 16)] = s
            return 0
        lax.fori_loop(0, _LAYERS + 1, esum, 0)
        pltpu.sync_copy(errt, errh)


@functools.cache
def _build_sc_kernel():
  mesh = plsc.VectorSubcoreMesh(core_axis_name="c", subcore_axis_name="s",
                                num_cores=1, num_subcores=_NS)
  return functools.partial(
    pl.kernel,
    out_type=(jax.ShapeDtypeStruct((_NPAD,), f32),
              jax.ShapeDtypeStruct((_ERRW,), f32)),
    mesh=mesh,
    compiler_params=pltpu.CompilerParams(needs_layout_passes=False),
    scratch_types=[
        pltpu.VMEM_SHARED((_NPAD,), f32),
        pltpu.VMEM_SHARED((_NPAD,), f32),
        pltpu.VMEM_SHARED((_NS * _ERRW,), f32),
        pltpu.VMEM((_W,), f32),
        pltpu.VMEM((_W,), f32),
        pltpu.VMEM((_W,), f32),
        pltpu.VMEM((_W,), f32),
        pltpu.VMEM((_PTILE,), i32),
        pltpu.VMEM((_PTILE,), f32),
        pltpu.VMEM((_PTILE,), f32),
        pltpu.VMEM((_PTILE,), f32),
        pltpu.VMEM((_NPAD,), f32),
        pltpu.VMEM((_CROWS, 128), i32),
        pltpu.VMEM((_CROWS, 128), i32),
        pltpu.VMEM((_CROWS, 128), i32),
        pltpu.VMEM((_CROWS, 128), f32),
        pltpu.VMEM((_CROWS, 128), f32),
        pltpu.VMEM((_CROWS, 128), f32),
        pltpu.VMEM((_ERRW,), f32),
        pltpu.VMEM((_NS * _ERRW,), f32),
        pltpu.VMEM((_ERRW,), f32),
        pltpu.SemaphoreType.DMA,
        pltpu.SemaphoreType.DMA,
    ],
  )(_sc_body)


def kernel(x, y, edge_index_no_diag, edge_attr_no_diag, edge_index, edge_attr,
           ybus):
    del y
    x0 = jnp.pad(x[:, 0], (0, _NPAD - _N))
    x1 = jnp.pad(x[:, 1], (0, _NPAD - _N))
    eye = jnp.eye(_NBUS, dtype=f32)
    dg = (ybus * eye).sum(axis=2).reshape(-1)
    dp = jnp.pad(dg, (0, _NPAD - _N))

    def prep(ei, ea):
        s = jnp.pad(ei[0].astype(i32), (0, _EPAD - _E)).reshape(_ROWS, 128)
        d = jnp.pad(ei[1].astype(i32), (0, _EPAD - _E)).reshape(_ROWS, 128)
        w = jnp.pad(ea.astype(f32), (0, _EPAD - _E)).reshape(_ROWS, 128)
        return s, d, w

    s1, d1, w1 = prep(edge_index_no_diag, edge_attr_no_diag)
    s2, d2, w2 = prep(edge_index, edge_attr)
    outp, errs = _build_sc_kernel()(x0, x1, dp, s1, d1, w1, s2, d2, w2)
    out = outp[:_N].reshape(_N, 1)
    return (out, *(errs[k * 16:(k + 1) * 16].sum() for k in range(_LAYERS + 1)))


# speedup vs baseline: 1.4968x; 1.0002x over previous
import functools

import jax
import jax.numpy as jnp
from jax import lax
from jax.experimental import pallas as pl
from jax.experimental.pallas import tpu as pltpu
from jax.experimental.pallas import tpu_sc as plsc

f32 = jnp.float32
i32 = jnp.int32

_N = 50000
_NBUS = 500
_NGRAPH = 100
_E = 800000
_LAYERS = 10

_NS = 16
_PTILE = 3136
_NPAD = _PTILE * _NS
_MARGIN = 512
_W = _PTILE + _MARGIN
_W2 = 3712

_CROWS = 56
_NCHUNK = 7
_RPT = _CROWS * _NCHUNK
_ROWS = _RPT * _NS
_EPAD = _ROWS * 128
_YTOT = _NGRAPH * _NBUS * _NBUS
_ERRW = (_LAYERS + 1) * 16


def _sc_body(x0h, x1h, dph, s1h, d1h, w1h, s2h, d2h, w2h,
             outh, errh,
             th_sh, ag_sh, erra_sh,
             pw, invdw, tb, agw, refx, outs, maskf, zb, threp,
             srcb, dstb0, dstb1, wb, msgb0, msgb1, errb, erracc, errt,
             sem_l, sem_s):
    wid = lax.axis_index("s")
    lo = pl.multiple_of(wid * _PTILE, 8)
    sw = pl.multiple_of(jnp.maximum(lo - _MARGIN, 0), 8)
    own = lo - sw
    tbase = wid * _RPT

    pltpu.sync_copy(x0h.at[pl.ds(sw, _W)], agw)
    pltpu.sync_copy(x1h.at[pl.ds(sw, _W)], tb)

    def pinit(v, _):
        pw[pl.ds(v * 16, 16)] = agw[pl.ds(v * 16, 16)] - tb[pl.ds(v * 16, 16)]
        return 0
    lax.fori_loop(0, _W // 16, pinit, 0)

    pltpu.sync_copy(dph.at[pl.ds(sw, _W)], tb)

    def dinit(v, _):
        d = tb[pl.ds(v * 16, 16)] * 100.0
        nz = d != 0.0
        dsafe = jnp.where(nz, d, 1.0)
        invdw[pl.ds(v * 16, 16)] = jnp.where(nz, 1.0 / dsafe, 0.0)
        return 0
    lax.fori_loop(0, _W // 16, dinit, 0)

    def minit(v, _):
        iv = invdw[pl.ds(own + v * 16, 16)]
        maskf[pl.ds(v * 16, 16)] = jnp.where(iv != 0.0, 1.0, 0.0)
        zb[pl.ds(v * 16, 16)] = jnp.zeros((16,), f32)
        i = lo + v * 16 + lax.iota(i32, 16)
        g = ((i.astype(f32) + 0.5) * (1.0 / 500.0)).astype(i32)
        g = jnp.minimum(g, _NGRAPH - 1)
        refx[pl.ds(v * 16, 16)] = g * _NBUS - sw
        return 0
    lax.fori_loop(0, _PTILE // 16, minit, 0)

    pltpu.sync_copy(zb, ag_sh.at[pl.ds(lo, _PTILE)])

    def einit(kk, _):
        errb[pl.ds(kk * 16, 16)] = jnp.zeros((16,), f32)
        return 0
    lax.fori_loop(0, _LAYERS + 1, einit, 0)
    plsc.subcore_barrier()

    _SCAT_BYTES = _CROWS * 128 * 4

    def edge_pass(sh, dh, wh):
        pltpu.sync_copy(th_sh, threp)

        def chunk(c, _):
            @pl.when(c > 0)
            def _():
                pltpu.make_async_copy(wh.at[pl.ds(0, _CROWS)], msgb0, sem_s
                                      ).wait()

            def do_chunk(dstx, msgx):
                rb = pl.multiple_of(tbase + c * _CROWS, 8)
                l1 = pltpu.async_copy(sh.at[pl.ds(rb, _CROWS)], srcb, sem_l)
                l2 = pltpu.async_copy(dh.at[pl.ds(rb, _CROWS)], dstx, sem_l)
                l3 = pltpu.async_copy(wh.at[pl.ds(rb, _CROWS)], wb, sem_l)
                l1.wait()
                l2.wait()
                l3.wait()

                def mrow(r, _):
                    for o in range(8):
                        sl = pl.ds(o * 16, 16)
                        th = plsc.load_gather(threp, [srcb[r, sl]])
                        msgx[r, sl] = th * wb[r, sl]
                    return 0
                lax.fori_loop(0, _CROWS, mrow, 0)
                for j in range(_CROWS):
                    pltpu.async_copy(msgx.at[j], ag_sh.at[dstx.at[j]], sem_s,
                                     add=True)

            @pl.when(c % 2 == 0)
            def _():
                do_chunk(dstb0, msgb0)

            @pl.when(c % 2 == 1)
            def _():
                do_chunk(dstb1, msgb1)
            return 0
        lax.fori_loop(0, _NCHUNK, chunk, 0)
        pltpu.make_async_copy(wh.at[pl.ds(0, _CROWS)], msgb0, sem_s).wait()

    def step(k, _):
        @pl.when(k > 0)
        def _():
            edge_pass(s1h, d1h, w1h)
        plsc.subcore_barrier()

        pltpu.sync_copy(ag_sh.at[pl.ds(sw, _W)], agw)
        plsc.subcore_barrier()
        pltpu.sync_copy(zb, ag_sh.at[pl.ds(lo, _PTILE)])

        def tcomp(v, _):
            sl = pl.ds(v * 16, 16)
            tb[sl] = (pw[sl] - agw[sl]) * invdw[sl]
            return 0
        lax.fori_loop(0, _W // 16, tcomp, 0)

        def ocomp(v, _):
            sl = pl.ds(v * 16, 16)
            t = tb[pl.ds(own + v * 16, 16)]
            tr = plsc.load_gather(tb, [refx[sl]])
            outs[sl] = (t - tr) * maskf[sl] * 100.0
            return 0
        lax.fori_loop(0, _PTILE // 16, ocomp, 0)

        pltpu.sync_copy(outs, th_sh.at[pl.ds(lo, _PTILE)])

        @pl.when(k == _LAYERS)
        def _():
            def fcomp(v, _):
                sl = pl.ds(v * 16, 16)
                tb[sl] = outs[sl] * 0.01
                return 0
            lax.fori_loop(0, _PTILE // 16, fcomp, 0)
            pltpu.sync_copy(tb.at[pl.ds(0, _PTILE)], outh.at[pl.ds(lo, _PTILE)])
        plsc.subcore_barrier()

        edge_pass(s2h, d2h, w2h)
        plsc.subcore_barrier()

        pltpu.sync_copy(ag_sh.at[pl.ds(lo, _PTILE)], agw.at[pl.ds(0, _PTILE)])

        def ecomp(v, acc):
            e = pw[pl.ds(own + v * 16, 16)] - agw[pl.ds(v * 16, 16)]
            return acc + jnp.abs(e)
        acc = lax.fori_loop(0, _PTILE // 16, ecomp, jnp.zeros((16,), f32))
        errb[pl.ds(k * 16, 16)] = acc
        pltpu.sync_copy(zb, ag_sh.at[pl.ds(lo, _PTILE)])
        plsc.subcore_barrier()
        return 0
    lax.fori_loop(0, _LAYERS + 1, step, 0)

    pltpu.sync_copy(errb, erra_sh.at[pl.ds(pl.multiple_of(wid * _ERRW, 8), _ERRW)])
    plsc.subcore_barrier()

    @pl.when(wid == 0)
    def _():
        pltpu.sync_copy(erra_sh, erracc)

        def esum(kk, _):
            s = jnp.zeros((16,), f32)
            for t in range(_NS):
                s = s + erracc[pl.ds(t * _ERRW + kk * 16, 16)]
            errt[pl.ds(kk * 16, 16)] = s
            return 0
        lax.fori_loop(0, _LAYERS + 1, esum, 0)
        pltpu.sync_copy(errt, errh)


@functools.cache
def _build_sc_kernel():
  mesh = plsc.VectorSubcoreMesh(core_axis_name="c", subcore_axis_name="s",
                                num_cores=1, num_subcores=_NS)
  return functools.partial(
    pl.kernel,
    out_type=(jax.ShapeDtypeStruct((_NPAD,), f32),
              jax.ShapeDtypeStruct((_ERRW,), f32)),
    mesh=mesh,
    compiler_params=pltpu.CompilerParams(needs_layout_passes=False),
    scratch_types=[
        pltpu.VMEM_SHARED((_NPAD,), f32),
        pltpu.VMEM_SHARED((_NPAD,), f32),
        pltpu.VMEM_SHARED((_NS * _ERRW,), f32),
        pltpu.VMEM((_W,), f32),
        pltpu.VMEM((_W,), f32),
        pltpu.VMEM((_W,), f32),
        pltpu.VMEM((_W,), f32),
        pltpu.VMEM((_PTILE,), i32),
        pltpu.VMEM((_PTILE,), f32),
        pltpu.VMEM((_PTILE,), f32),
        pltpu.VMEM((_PTILE,), f32),
        pltpu.VMEM((_NPAD,), f32),
        pltpu.VMEM((_CROWS, 128), i32),
        pltpu.VMEM((_CROWS, 128), i32),
        pltpu.VMEM((_CROWS, 128), i32),
        pltpu.VMEM((_CROWS, 128), f32),
        pltpu.VMEM((_CROWS, 128), f32),
        pltpu.VMEM((_CROWS, 128), f32),
        pltpu.VMEM((_ERRW,), f32),
        pltpu.VMEM((_NS * _ERRW,), f32),
        pltpu.VMEM((_ERRW,), f32),
        pltpu.SemaphoreType.DMA,
        pltpu.SemaphoreType.DMA,
    ],
  )(_sc_body)


def kernel(x, y, edge_index_no_diag, edge_attr_no_diag, edge_index, edge_attr,
           ybus):
    del y
    x0 = jnp.pad(x[:, 0], (0, _NPAD - _N))
    x1 = jnp.pad(x[:, 1], (0, _NPAD - _N))
    eye = jnp.eye(_NBUS, dtype=f32)
    dg = (ybus * eye).sum(axis=2).reshape(-1)
    dp = jnp.pad(dg, (0, _NPAD - _N))

    def prep(ei, ea):
        s = jnp.pad(ei[0].astype(i32), (0, _EPAD - _E)).reshape(_ROWS, 128)
        d = jnp.pad(ei[1].astype(i32), (0, _EPAD - _E)).reshape(_ROWS, 128)
        w = jnp.pad(ea.astype(f32), (0, _EPAD - _E)).reshape(_ROWS, 128)
        return s, d, w

    s1, d1, w1 = prep(edge_index_no_diag, edge_attr_no_diag)
    s2, d2, w2 = prep(edge_index, edge_attr)
    outp, errs = _build_sc_kernel()(x0, x1, dp, s1, d1, w1, s2, d2, w2)
    out = outp[:_N].reshape(_N, 1)
    return (out, *(errs[k * 16:(k + 1) * 16].sum() for k in range(_LAYERS + 1)))
